# baseline (speedup 1.0000x reference)
import jax
import jax.numpy as jnp
from jax import lax
from jax.experimental import pallas as pl
from jax.experimental.pallas import tpu as pltpu
from jax.experimental.pallas import tpu_sc as plsc

N = 10000
E = 320000
DX = 64
DG = 64
H = 128
OUT = 2
PO_LEN = 32768

NC = 2
NS = 16
NW = NC * NS

CH = 128
NCHUNK = E // CH
NPT = 80
NBI = 16
ECAP = NW * NPT * CH
NP = 10240
ROWS_T = NP // NS
DW = 128

BR = 1000
NB = N // BR

_f32 = jnp.float32


def _seg_body(p_hbm, src2d, dst2d, z128_hbm,
              seg_out, acc, isall, idall, rows0, rows1, g0, g1, s0, s1):
    c = lax.axis_index("c")
    s = lax.axis_index("s")
    wid = s * NC + c
    r0 = s * ROWS_T
    pltpu.sync_copy(z128_hbm.at[pl.ds(r0, ROWS_T)], acc.at[pl.ds(r0, ROWS_T)])
    start = wid * NPT
    cnt = jnp.minimum(jnp.maximum(NCHUNK - start, 0), NPT)
    plsc.subcore_barrier()

    def gat(l, rows, sem):
        pltpu.async_copy(p_hbm.at[isall.at[l]], rows, sem)

    def wait_gat(l, rows, sem):
        pltpu.make_async_copy(p_hbm.at[isall.at[l]], rows, sem).wait()

    def sca(l, rows, sem):
        pltpu.async_copy(rows, acc.at[idall.at[l]], sem, add=True)

    def wait_sca(rows, sem):
        pltpu.make_async_copy(rows, acc.at[idall.at[0]], sem).wait()

    def blk(b, carry):
        j_lo = b * NBI

        @pl.when(j_lo < cnt)
        def _():
            @pl.when(b > 0)
            def _():
                wait_sca(rows1, s1)

            pltpu.sync_copy(src2d.at[pl.ds(start + j_lo, NBI)], isall)
            pltpu.sync_copy(dst2d.at[pl.ds(start + j_lo, NBI)], idall)
            gat(0, rows0, g0)

        def pair(i, carry2):
            j0 = j_lo + 2 * i
            j1 = j0 + 1
            l0 = 2 * i
            l1 = l0 + 1

            @pl.when(j0 < cnt)
            def _():
                wait_gat(l0, rows0, g0)

                @pl.when(l0 >= 1)
                def _():
                    wait_sca(rows1, s1)

                @pl.when(j1 < cnt)
                def _():
                    gat(l1, rows1, g1)

                sca(l0, rows0, s0)

            @pl.when(j1 < cnt)
            def _():
                wait_gat(l1, rows1, g1)
                wait_sca(rows0, s0)

                @pl.when((j1 + 1 < cnt) & (l1 + 1 < NBI))
                def _():
                    gat(l1 + 1, rows0, g0)

                sca(l1, rows1, s1)

            return carry2

        lax.fori_loop(0, NBI // 2, pair, 0)
        return carry

    lax.fori_loop(0, NPT // NBI, blk, 0)

    @pl.when(cnt >= 2)
    def _():
        wait_sca(rows1, s1)

    plsc.subcore_barrier()

    pltpu.sync_copy(acc.at[pl.ds(r0, ROWS_T)],
                    seg_out.at[c, pl.ds(r0, ROWS_T)])


def _make_seg_call(interpret=False):
    mesh = plsc.VectorSubcoreMesh(core_axis_name="c", subcore_axis_name="s",
                                  num_cores=NC, num_subcores=NS)
    return pl.kernel(
        _seg_body,
        out_type=[jax.ShapeDtypeStruct((NC, NP, H), _f32)],
        mesh=mesh,
        scratch_types=[
            pltpu.VMEM_SHARED((NP, H), _f32),
            pltpu.VMEM((NBI, CH), jnp.int32),
            pltpu.VMEM((NBI, CH), jnp.int32),
            pltpu.VMEM((CH, H), _f32),
            pltpu.VMEM((CH, H), _f32),
            pltpu.SemaphoreType.DMA,
            pltpu.SemaphoreType.DMA,
            pltpu.SemaphoreType.DMA,
            pltpu.SemaphoreType.DMA,
        ],
        interpret=interpret,
    )


def _deg_body(dst2d, z_hbm, ones_hbm, deg_out, acc, idall, ones_v, s0, s1):
    c = lax.axis_index("c")
    s = lax.axis_index("s")
    wid = s * NC + c
    r0 = s * ROWS_T
    pltpu.sync_copy(z_hbm.at[pl.ds(r0, ROWS_T)], acc.at[pl.ds(r0, ROWS_T)])
    pltpu.sync_copy(ones_hbm, ones_v)
    start = wid * NPT
    cnt = jnp.minimum(jnp.maximum(NCHUNK - start, 0), NPT)
    plsc.subcore_barrier()

    def sca(l, sem):
        pltpu.async_copy(ones_v, acc.at[idall.at[l]], sem, add=True)

    def wait_sca(sem):
        pltpu.make_async_copy(ones_v, acc.at[idall.at[0]], sem).wait()

    def blk(b, carry):
        j_lo = b * NBI

        @pl.when(j_lo < cnt)
        def _():
            @pl.when(b > 0)
            def _():
                wait_sca(s0)
                wait_sca(s1)

            pltpu.sync_copy(dst2d.at[pl.ds(start + j_lo, NBI)], idall)

        def pairs(i, carry2):
            l0 = 2 * i
            l1 = l0 + 1

            @pl.when(j_lo + l0 < cnt)
            def _():
                @pl.when(l0 >= 2)
                def _():
                    wait_sca(s0)

                sca(l0, s0)

            @pl.when(j_lo + l1 < cnt)
            def _():
                @pl.when(l1 >= 2)
                def _():
                    wait_sca(s1)

                sca(l1, s1)

            return carry2

        lax.fori_loop(0, NBI // 2, pairs, 0)
        return carry

    lax.fori_loop(0, NPT // NBI, blk, 0)

    @pl.when(cnt >= 2)
    def _():
        wait_sca(s0)
        wait_sca(s1)

    plsc.subcore_barrier()
    pltpu.sync_copy(acc.at[pl.ds(r0, ROWS_T)],
                    deg_out.at[c, pl.ds(r0, ROWS_T)])


def _make_deg_call(interpret=False):
    mesh = plsc.VectorSubcoreMesh(core_axis_name="c", subcore_axis_name="s",
                                  num_cores=NC, num_subcores=NS)
    return pl.kernel(
        _deg_body,
        out_type=[jax.ShapeDtypeStruct((NC, NP, DW), _f32)],
        mesh=mesh,
        scratch_types=[
            pltpu.VMEM_SHARED((NP, DW), _f32),
            pltpu.VMEM((NBI, CH), jnp.int32),
            pltpu.VMEM((CH, DW), _f32),
            pltpu.SemaphoreType.DMA,
            pltpu.SemaphoreType.DMA,
        ],
        interpret=interpret,
    )


def _po_gather_body(h_hbm, po2d, out_hbm, idx, rows0, rows1, sem0, sem1):
    c = lax.axis_index("c")
    s = lax.axis_index("s")
    wid = s * NC + c
    per_tile = PO_LEN // NW
    nch = per_tile // CH
    base = wid * per_tile
    pltpu.sync_copy(po2d.at[pl.ds(wid * nch, nch)], idx)

    def gather(j, rows, sem):
        return pltpu.async_copy(h_hbm.at[idx.at[j]], rows, sem)

    gather(0, rows0, sem0)

    def pair(i, carry):
        j0 = 2 * i
        j1 = j0 + 1
        gather(j1, rows1, sem1)
        pltpu.make_async_copy(h_hbm.at[idx.at[j0]], rows0, sem0).wait()
        pltpu.sync_copy(rows0, out_hbm.at[pl.ds(base + j0 * CH, CH)])

        @pl.when(j1 + 1 < nch)
        def _():
            gather(j1 + 1, rows0, sem0)

        pltpu.make_async_copy(h_hbm.at[idx.at[j1]], rows1, sem1).wait()
        pltpu.sync_copy(rows1, out_hbm.at[pl.ds(base + j1 * CH, CH)])
        return carry

    lax.fori_loop(0, nch // 2, pair, 0)


def _make_po_gather(interpret=False):
    mesh = plsc.VectorSubcoreMesh(core_axis_name="c", subcore_axis_name="s",
                                  num_cores=NC, num_subcores=NS)
    return pl.kernel(
        _po_gather_body,
        out_type=jax.ShapeDtypeStruct((PO_LEN, H), _f32),
        mesh=mesh,
        scratch_types=[
            pltpu.VMEM((PO_LEN // NW // CH, CH), jnp.int32),
            pltpu.VMEM((CH, H), _f32),
            pltpu.VMEM((CH, H), _f32),
            pltpu.SemaphoreType.DMA,
            pltpu.SemaphoreType.DMA,
        ],
        interpret=interpret,
    )


def _bn_phases(ph, j, t_fn, out_fn, g_r, be_r, tbuf, s1, s2):
    @pl.when(ph == 0)
    def _():
        t = t_fn()
        tbuf[pl.ds(j * BR, BR), :] = t

        @pl.when(j == 0)
        def _():
            s1[...] = jnp.zeros((1, H), _f32)
            s2[...] = jnp.zeros((1, H), _f32)

        s1[...] += jnp.sum(t, axis=0, keepdims=True)
        s2[...] += jnp.sum(t * t, axis=0, keepdims=True)

    @pl.when(ph == 1)
    def _():
        mu = s1[...] * (1.0 / N)
        var = s2[...] * (1.0 / N) - mu * mu
        t = tbuf[pl.ds(j * BR, BR), :]
        hn = (t - mu) * lax.rsqrt(var + 1e-5) * g_r[...] + be_r[...]
        out_fn(jnp.maximum(hn, 0.0))


def _layer0_body(sa0_r, sa1_r, sb0_r, sb1_r, dp0_r, dp1_r,
                 x_r, g0_r, g1_r, g2_r, g_r, be_r, wl_r, wr_r, b_r,
                 deg_o, h_o, tbuf, s1, s2):
    ph = pl.program_id(0)
    j = pl.program_id(1)
    deg = jnp.maximum(dp0_r[0, :, 0:1] + dp1_r[0, :, 0:1], 1.0)
    deg_o[...] = deg

    def t_fn():
        agg_a = (sa0_r[0] + sa1_r[0]) / deg
        agg_b = (sb0_r[0] + sb1_r[0]) / deg
        return (jnp.dot(agg_a, wl_r[pl.ds(0, H), :])
                + jnp.dot(agg_b, wl_r[pl.ds(H, H), :])
                + jnp.dot(x_r[...], wr_r[pl.ds(0, DX), :])
                + jnp.dot(g0_r[...], wr_r[pl.ds(DX, DG), :])
                + jnp.dot(g1_r[...], wr_r[pl.ds(DX + DG, DG), :])
                + jnp.dot(g2_r[...], wr_r[pl.ds(DX + 2 * DG, DG), :])
                + b_r[...])

    def out_fn(hn):
        h_o[...] = hn

    _bn_phases(ph, j, t_fn, out_fn, g_r, be_r, tbuf, s1, s2)


def _layer_body(s0_r, s1g_r, deg_r, h_r, g_r, be_r, wl_r, wr_r, b_r,
                h_o, tbuf, s1, s2):
    ph = pl.program_id(0)
    j = pl.program_id(1)

    def t_fn():
        agg = (s0_r[0] + s1g_r[0]) / jnp.maximum(deg_r[...], 1.0)
        return (jnp.dot(agg, wl_r[...]) + jnp.dot(h_r[...], wr_r[...])
                + b_r[...])

    def out_fn(hn):
        h_o[...] = hn

    _bn_phases(ph, j, t_fn, out_fn, g_r, be_r, tbuf, s1, s2)


def _j0(p, j):
    return jnp.where(p == 0, j, 0)


def _seg_specs():
    return [pl.BlockSpec((1, BR, H), lambda p, j: (0, _j0(p, j), 0)),
            pl.BlockSpec((1, BR, H), lambda p, j: (1, _j0(p, j), 0))]


_SCRATCH = [pltpu.VMEM((N, H), _f32),
            pltpu.VMEM((1, H), _f32),
            pltpu.VMEM((1, H), _f32)]


def _layer0_call(segA, segB, degp, x, g0, g1, g2, g, be, wl, wr, b,
                 interpret=False):
    vspec = pl.BlockSpec((1, H), lambda p, j: (0, 0))
    din = DX + 3 * DG
    return pl.pallas_call(
        _layer0_body,
        grid=(2, NB),
        in_specs=(_seg_specs() + _seg_specs()
                  + [pl.BlockSpec((1, BR, DW), lambda p, j: (0, _j0(p, j), 0)),
                     pl.BlockSpec((1, BR, DW), lambda p, j: (1, _j0(p, j), 0)),
                     pl.BlockSpec((BR, DX), lambda p, j: (_j0(p, j), 0)),
                     pl.BlockSpec((BR, DG), lambda p, j: (_j0(p, j), 0)),
                     pl.BlockSpec((BR, DG), lambda p, j: (_j0(p, j), 0)),
                     pl.BlockSpec((BR, DG), lambda p, j: (_j0(p, j), 0)),
                     vspec, vspec,
                     pl.BlockSpec((din, H), lambda p, j: (0, 0)),
                     pl.BlockSpec((din, H), lambda p, j: (0, 0)),
                     vspec]),
        out_specs=[
            pl.BlockSpec((BR, 1), lambda p, j: (_j0(p, j), 0)),
            pl.BlockSpec((BR, H), lambda p, j: (j, 0)),
        ],
        out_shape=[jax.ShapeDtypeStruct((N, 1), _f32),
                   jax.ShapeDtypeStruct((N, H), _f32)],
        scratch_shapes=_SCRATCH,
        interpret=interpret,
    )(segA, segA, segB, segB, degp, degp, x, g0, g1, g2,
      g.reshape(1, H), be.reshape(1, H), wl, wr, b.reshape(1, H))


def _layer_call(segp, deg, h, g, be, wl, wr, b, interpret=False):
    vspec = pl.BlockSpec((1, H), lambda p, j: (0, 0))
    wspec = pl.BlockSpec((H, H), lambda p, j: (0, 0))
    return pl.pallas_call(
        _layer_body,
        grid=(2, NB),
        in_specs=(_seg_specs()
                  + [pl.BlockSpec((BR, 1), lambda p, j: (_j0(p, j), 0)),
                     pl.BlockSpec((BR, H), lambda p, j: (_j0(p, j), 0)),
                     vspec, vspec, wspec, wspec, vspec]),
        out_specs=[pl.BlockSpec((BR, H), lambda p, j: (j, 0))],
        out_shape=[jax.ShapeDtypeStruct((N, H), _f32)],
        scratch_shapes=_SCRATCH,
        interpret=interpret,
    )(segp, segp, deg, h, g.reshape(1, H), be.reshape(1, H),
      wl, wr, b.reshape(1, H))


BRH = 512
NBH = PO_LEN // BRH


def _head1_body(arr_r, bd_r, b1t_r, cd_r, b2_r, v_o):
    y = jnp.maximum(
        jnp.dot(arr_r[...].astype(jnp.bfloat16), bd_r[...],
                preferred_element_type=_f32) + b1t_r[...], 0.0)
    zv = jnp.dot(y.astype(jnp.bfloat16), cd_r[...],
                 preferred_element_type=_f32) + b2_r[...]
    zv3 = zv.reshape(BRH // 8, 8, 16)
    v_o[...] = jnp.concatenate([zv3[:, u, :] for u in range(8)], axis=1)


def _head1_call(arr, w1, b1, w2, b2, interpret=False):
    import jax.scipy.linalg as jsl
    bd = jsl.block_diag(*([w1] * 16)).astype(jnp.bfloat16)
    cd = jsl.block_diag(*([w2] * 16)).astype(jnp.bfloat16)
    b1t = jnp.tile(b1, (16,)).reshape(1, 16 * H)
    return pl.pallas_call(
        _head1_body,
        grid=(NBH,),
        in_specs=[
            pl.BlockSpec((BRH, H), lambda j: (j, 0)),
            pl.BlockSpec((H, 16 * H), lambda j: (0, 0)),
            pl.BlockSpec((1, 16 * H), lambda j: (0, 0)),
            pl.BlockSpec((16 * H, 16), lambda j: (0, 0)),
            pl.BlockSpec((1, 1), lambda j: (0, 0)),
        ],
        out_specs=[pl.BlockSpec((BRH // 8, H), lambda j: (j, 0))],
        out_shape=[jax.ShapeDtypeStruct((PO_LEN // 8, H), _f32)],
        interpret=interpret,
    )(arr, bd, b1t, cd, b2.reshape(1, 1))


def _head2_body(v_r, g_r, be_r, w1_r, b1_r, w2_r, b2_r, o_r):
    v = v_r[...]
    mu = jnp.mean(v, axis=0, keepdims=True)
    d = v - mu
    var = jnp.mean(d * d, axis=0, keepdims=True)
    f = jnp.maximum(d * lax.rsqrt(var + 1e-5) * g_r[...] + be_r[...], 0.0)
    u = jnp.maximum(jnp.dot(f, w1_r[...]) + b1_r[...], 0.0)
    o_r[...] = jnp.maximum(jnp.dot(u, w2_r[...]) + b2_r[...], 0.0)


def _head2_call(v, g, be, w1, b1, w2, b2, interpret=False):
    m = PO_LEN // 8
    return pl.pallas_call(
        _head2_body,
        out_shape=jax.ShapeDtypeStruct((m, OUT), _f32),
        interpret=interpret,
    )(v, g.reshape(1, H), be.reshape(1, H), w1, b1.reshape(1, H),
      w2, b2.reshape(1, OUT))


def kernel(x, gam0, gam1, gam2, edge_index, po, po_batch,
           conv0_Wl, conv0_Wr, conv0_b, bn0_g, bn0_b,
           conv1_Wl, conv1_Wr, conv1_b, bn1_g, bn1_b,
           conv2_Wl, conv2_Wr, conv2_b, bn2_g, bn2_b,
           conv3_Wl, conv3_Wr, conv3_b, bn3_g, bn3_b,
           mlp1_W1, mlp1_b1, mlp1_W2, mlp1_b2,
           bnf_g, bnf_b,
           mlp2_W1, mlp2_b1, mlp2_W2, mlp2_b2):
    pad = jnp.zeros((ECAP - E,), jnp.int32)
    src2d = jnp.concatenate([edge_index[0], pad]).reshape(ECAP // CH, CH)
    dst2d = jnp.concatenate([edge_index[1], pad]).reshape(ECAP // CH, CH)
    po2d = po.reshape(PO_LEN // CH, CH)
    z128 = jnp.zeros((NP, H), _f32)
    ones_rows = jnp.ones((CH, DW), _f32)
    h0a = jnp.concatenate([x, gam0], axis=1)
    h0b = jnp.concatenate([gam1, gam2], axis=1)

    seg_call = _make_seg_call()
    deg_call = _make_deg_call()
    po_call = _make_po_gather()

    degp = deg_call(dst2d, z128, ones_rows)[0]

    segA = seg_call(h0a, src2d, dst2d, z128)[0]
    segB = seg_call(h0b, src2d, dst2d, z128)[0]
    deg, h1 = _layer0_call(segA, segB, degp, x, gam0, gam1, gam2,
                           bn0_g, bn0_b, conv0_Wl, conv0_Wr, conv0_b)
    segp = seg_call(h1, src2d, dst2d, z128)[0]
    h2 = _layer_call(segp, deg, h1, bn1_g, bn1_b,
                     conv1_Wl, conv1_Wr, conv1_b)[0]
    segp = seg_call(h2, src2d, dst2d, z128)[0]
    h3 = _layer_call(segp, deg, h2, bn2_g, bn2_b,
                     conv2_Wl, conv2_Wr, conv2_b)[0]
    segp = seg_call(h3, src2d, dst2d, z128)[0]
    h4 = _layer_call(segp, deg, h3, bn3_g, bn3_b,
                     conv3_Wl, conv3_Wr, conv3_b)[0]

    arr = po_call(h4, po2d)
    v = _head1_call(arr, mlp1_W1, mlp1_b1, mlp1_W2, mlp1_b2)[0]
    return _head2_call(v, bnf_g, bnf_b, mlp2_W1, mlp2_b1, mlp2_W2, mlp2_b2)


# speedup vs baseline: 1.1295x; 1.1295x over previous
import jax
import jax.numpy as jnp
from jax import lax
from jax.experimental import pallas as pl
from jax.experimental.pallas import tpu as pltpu
from jax.experimental.pallas import tpu_sc as plsc

N = 10000
E = 320000
DX = 64
DG = 64
H = 128
OUT = 2
PO_LEN = 32768

NC = 2
NS = 16
NW = NC * NS

CH = 128
NCHUNK = E // CH
NPT = 80
NBI = 16
ECAP = NW * NPT * CH
NP = 10240
ROWS_T = NP // NS
DW = 128

BR = 1000
NB = N // BR

_f32 = jnp.float32


def _seg_body(p_hbm, src2d, dst2d, z128_hbm,
              seg_out, acc, isA, idA, isB, idB, rows0, rows1,
              sem0, sem1, semiA, semiB):
    c = lax.axis_index("c")
    s = lax.axis_index("s")
    wid = s * NC + c

    r0 = s * ROWS_T
    pltpu.sync_copy(z128_hbm.at[pl.ds(r0, ROWS_T)], acc.at[pl.ds(r0, ROWS_T)])
    start = wid * NPT
    cnt = jnp.minimum(jnp.maximum(NCHUNK - start, 0), NPT)
    nblk = NPT // NBI
    pairs = [(isA, idA, semiA), (isB, idB, semiB)]

    def gather(ix, j, rows, sem):
        return pltpu.async_copy(p_hbm.at[ix.at[j]], rows, sem)

    def wait_gather(ix, j, rows, sem):
        pltpu.make_async_copy(p_hbm.at[ix.at[j]], rows, sem).wait()

    def idx_load(b, ix, idx_, semi, sync):
        sl = pl.ds(start + b * NBI, NBI)
        if sync:
            pltpu.sync_copy(src2d.at[sl], ix)
            pltpu.sync_copy(dst2d.at[sl], idx_)
        else:
            pltpu.async_copy(src2d.at[sl], ix, semi)
            pltpu.async_copy(dst2d.at[sl], idx_, semi)

    def idx_wait(b, ix, idx_, semi):
        sl = pl.ds(start + b * NBI, NBI)
        pltpu.make_async_copy(src2d.at[sl], ix, semi).wait()
        pltpu.make_async_copy(dst2d.at[sl], idx_, semi).wait()

    idx_load(0, isA, idA, semiA, sync=True)
    gather(isA, 0, rows0, sem0)

    @pl.when(NBI < cnt)
    def _():
        idx_load(1, isB, idB, semiB, sync=False)

    plsc.subcore_barrier()

    for b in range(nblk):
        ix, idx_, semi = pairs[b % 2]
        j_lo = b * NBI

        if b >= 1:
            @pl.when(j_lo < cnt)
            def _(b=b, ix=ix, idx_=idx_, semi=semi, j_lo=j_lo):
                idx_wait(b, ix, idx_, semi)
                gather(ix, 0, rows0, sem0)
                if b + 1 < nblk:
                    nix, nidx, nsemi = pairs[(b + 1) % 2]

                    @pl.when(j_lo + NBI < cnt)
                    def _():
                        idx_load(b + 1, nix, nidx, nsemi, sync=False)

        for i in range(NBI // 2):
            j0 = j_lo + 2 * i
            j1 = j0 + 1

            @pl.when(j0 < cnt)
            def _(i=i, j0=j0, j1=j1, ix=ix, idx_=idx_):
                @pl.when(j1 < cnt)
                def _():
                    gather(ix, 2 * i + 1, rows1, sem1)

                wait_gather(ix, 2 * i, rows0, sem0)
                pltpu.sync_copy(rows0, acc.at[idx_.at[2 * i]], add=True)

            @pl.when(j1 < cnt)
            def _(i=i, j1=j1, ix=ix, idx_=idx_):
                if 2 * i + 2 < NBI:
                    @pl.when(j1 + 1 < cnt)
                    def _():
                        gather(ix, 2 * i + 2, rows0, sem0)

                wait_gather(ix, 2 * i + 1, rows1, sem1)
                pltpu.sync_copy(rows1, acc.at[idx_.at[2 * i + 1]], add=True)

    plsc.subcore_barrier()

    pltpu.sync_copy(acc.at[pl.ds(r0, ROWS_T)],
                    seg_out.at[c, pl.ds(r0, ROWS_T)])


def _make_seg_call(interpret=False):
    mesh = plsc.VectorSubcoreMesh(core_axis_name="c", subcore_axis_name="s",
                                  num_cores=NC, num_subcores=NS)
    return pl.kernel(
        _seg_body,
        out_type=[jax.ShapeDtypeStruct((NC, NP, H), _f32)],
        mesh=mesh,
        scratch_types=[
            pltpu.VMEM_SHARED((NP, H), _f32),
            pltpu.VMEM((NBI, CH), jnp.int32),
            pltpu.VMEM((NBI, CH), jnp.int32),
            pltpu.VMEM((NBI, CH), jnp.int32),
            pltpu.VMEM((NBI, CH), jnp.int32),
            pltpu.VMEM((CH, H), _f32),
            pltpu.VMEM((CH, H), _f32),
            pltpu.SemaphoreType.DMA,
            pltpu.SemaphoreType.DMA,
            pltpu.SemaphoreType.DMA,
            pltpu.SemaphoreType.DMA,
        ],
        interpret=interpret,
    )


def _deg_body(dst2d, z_hbm, ones_hbm, deg_out, acc, idA, idB, ones_v,
              semiA, semiB):
    c = lax.axis_index("c")
    s = lax.axis_index("s")
    wid = s * NC + c
    r0 = s * ROWS_T
    pltpu.sync_copy(z_hbm.at[pl.ds(r0, ROWS_T)], acc.at[pl.ds(r0, ROWS_T)])
    pltpu.sync_copy(ones_hbm, ones_v)
    start = wid * NPT
    cnt = jnp.minimum(jnp.maximum(NCHUNK - start, 0), NPT)
    nblk = NPT // NBI
    pairs = [(idA, semiA), (idB, semiB)]

    pltpu.sync_copy(dst2d.at[pl.ds(start, NBI)], idA)

    @pl.when(NBI < cnt)
    def _():
        pltpu.async_copy(dst2d.at[pl.ds(start + NBI, NBI)], idB, semiB)

    plsc.subcore_barrier()

    for b in range(nblk):
        idx_, semi = pairs[b % 2]
        j_lo = b * NBI

        if b >= 1:
            @pl.when(j_lo < cnt)
            def _(b=b, idx_=idx_, semi=semi, j_lo=j_lo):
                pltpu.make_async_copy(dst2d.at[pl.ds(start + b * NBI, NBI)],
                                      idx_, semi).wait()
                if b + 1 < nblk:
                    nidx, nsemi = pairs[(b + 1) % 2]

                    @pl.when(j_lo + NBI < cnt)
                    def _():
                        pltpu.async_copy(
                            dst2d.at[pl.ds(start + (b + 1) * NBI, NBI)],
                            nidx, nsemi)

        for i in range(NBI):
            @pl.when(j_lo + i < cnt)
            def _(i=i, idx_=idx_):
                pltpu.sync_copy(ones_v, acc.at[idx_.at[i]], add=True)

    plsc.subcore_barrier()
    pltpu.sync_copy(acc.at[pl.ds(r0, ROWS_T)],
                    deg_out.at[c, pl.ds(r0, ROWS_T)])


def _make_deg_call(interpret=False):
    mesh = plsc.VectorSubcoreMesh(core_axis_name="c", subcore_axis_name="s",
                                  num_cores=NC, num_subcores=NS)
    return pl.kernel(
        _deg_body,
        out_type=[jax.ShapeDtypeStruct((NC, NP, DW), _f32)],
        mesh=mesh,
        scratch_types=[
            pltpu.VMEM_SHARED((NP, DW), _f32),
            pltpu.VMEM((NBI, CH), jnp.int32),
            pltpu.VMEM((NBI, CH), jnp.int32),
            pltpu.VMEM((CH, DW), _f32),
            pltpu.SemaphoreType.DMA,
            pltpu.SemaphoreType.DMA,
        ],
        interpret=interpret,
    )


def _po_gather_body(h_hbm, po2d, out_hbm, idx, rows0, rows1, sem0, sem1):
    c = lax.axis_index("c")
    s = lax.axis_index("s")
    wid = s * NC + c
    per_tile = PO_LEN // NW
    nch = per_tile // CH
    base = wid * per_tile
    pltpu.sync_copy(po2d.at[pl.ds(wid * nch, nch)], idx)

    def gather(j, rows, sem):
        return pltpu.async_copy(h_hbm.at[idx.at[j]], rows, sem)

    gather(0, rows0, sem0)

    def pair(i, carry):
        j0 = 2 * i
        j1 = j0 + 1
        gather(j1, rows1, sem1)
        pltpu.make_async_copy(h_hbm.at[idx.at[j0]], rows0, sem0).wait()
        pltpu.sync_copy(rows0, out_hbm.at[pl.ds(base + j0 * CH, CH)])

        @pl.when(j1 + 1 < nch)
        def _():
            gather(j1 + 1, rows0, sem0)

        pltpu.make_async_copy(h_hbm.at[idx.at[j1]], rows1, sem1).wait()
        pltpu.sync_copy(rows1, out_hbm.at[pl.ds(base + j1 * CH, CH)])
        return carry

    lax.fori_loop(0, nch // 2, pair, 0)


def _make_po_gather(interpret=False):
    mesh = plsc.VectorSubcoreMesh(core_axis_name="c", subcore_axis_name="s",
                                  num_cores=NC, num_subcores=NS)
    return pl.kernel(
        _po_gather_body,
        out_type=jax.ShapeDtypeStruct((PO_LEN, H), _f32),
        mesh=mesh,
        scratch_types=[
            pltpu.VMEM((PO_LEN // NW // CH, CH), jnp.int32),
            pltpu.VMEM((CH, H), _f32),
            pltpu.VMEM((CH, H), _f32),
            pltpu.SemaphoreType.DMA,
            pltpu.SemaphoreType.DMA,
        ],
        interpret=interpret,
    )


def _bn_phases(ph, j, t_fn, out_fn, g_r, be_r, tbuf, s1, s2):
    @pl.when(ph == 0)
    def _():
        t = t_fn()
        tbuf[pl.ds(j * BR, BR), :] = t

        @pl.when(j == 0)
        def _():
            s1[...] = jnp.zeros((1, H), _f32)
            s2[...] = jnp.zeros((1, H), _f32)

        s1[...] += jnp.sum(t, axis=0, keepdims=True)
        s2[...] += jnp.sum(t * t, axis=0, keepdims=True)

    @pl.when(ph == 1)
    def _():
        mu = s1[...] * (1.0 / N)
        var = s2[...] * (1.0 / N) - mu * mu
        t = tbuf[pl.ds(j * BR, BR), :]
        hn = (t - mu) * lax.rsqrt(var + 1e-5) * g_r[...] + be_r[...]
        out_fn(jnp.maximum(hn, 0.0))


def _layer0_body(sa0_r, sa1_r, sb0_r, sb1_r, dp0_r, dp1_r,
                 x_r, g0_r, g1_r, g2_r, g_r, be_r, wl_r, wr_r, b_r,
                 deg_o, h_o, tbuf, s1, s2):
    ph = pl.program_id(0)
    j = pl.program_id(1)
    deg = jnp.maximum(dp0_r[0, :, 0:1] + dp1_r[0, :, 0:1], 1.0)
    deg_o[...] = deg

    def t_fn():
        agg_a = (sa0_r[0] + sa1_r[0]) / deg
        agg_b = (sb0_r[0] + sb1_r[0]) / deg
        return (jnp.dot(agg_a, wl_r[pl.ds(0, H), :])
                + jnp.dot(agg_b, wl_r[pl.ds(H, H), :])
                + jnp.dot(x_r[...], wr_r[pl.ds(0, DX), :])
                + jnp.dot(g0_r[...], wr_r[pl.ds(DX, DG), :])
                + jnp.dot(g1_r[...], wr_r[pl.ds(DX + DG, DG), :])
                + jnp.dot(g2_r[...], wr_r[pl.ds(DX + 2 * DG, DG), :])
                + b_r[...])

    def out_fn(hn):
        h_o[...] = hn

    _bn_phases(ph, j, t_fn, out_fn, g_r, be_r, tbuf, s1, s2)


def _layer_body(s0_r, s1g_r, deg_r, h_r, g_r, be_r, wl_r, wr_r, b_r,
                h_o, tbuf, s1, s2):
    ph = pl.program_id(0)
    j = pl.program_id(1)

    def t_fn():
        agg = (s0_r[0] + s1g_r[0]) / jnp.maximum(deg_r[...], 1.0)
        return (jnp.dot(agg, wl_r[...]) + jnp.dot(h_r[...], wr_r[...])
                + b_r[...])

    def out_fn(hn):
        h_o[...] = hn

    _bn_phases(ph, j, t_fn, out_fn, g_r, be_r, tbuf, s1, s2)


def _j0(p, j):
    return jnp.where(p == 0, j, 0)


def _seg_specs():
    return [pl.BlockSpec((1, BR, H), lambda p, j: (0, _j0(p, j), 0)),
            pl.BlockSpec((1, BR, H), lambda p, j: (1, _j0(p, j), 0))]


_SCRATCH = [pltpu.VMEM((N, H), _f32),
            pltpu.VMEM((1, H), _f32),
            pltpu.VMEM((1, H), _f32)]


def _layer0_call(segA, segB, degp, x, g0, g1, g2, g, be, wl, wr, b,
                 interpret=False):
    vspec = pl.BlockSpec((1, H), lambda p, j: (0, 0))
    din = DX + 3 * DG
    return pl.pallas_call(
        _layer0_body,
        grid=(2, NB),
        in_specs=(_seg_specs() + _seg_specs()
                  + [pl.BlockSpec((1, BR, DW), lambda p, j: (0, _j0(p, j), 0)),
                     pl.BlockSpec((1, BR, DW), lambda p, j: (1, _j0(p, j), 0)),
                     pl.BlockSpec((BR, DX), lambda p, j: (_j0(p, j), 0)),
                     pl.BlockSpec((BR, DG), lambda p, j: (_j0(p, j), 0)),
                     pl.BlockSpec((BR, DG), lambda p, j: (_j0(p, j), 0)),
                     pl.BlockSpec((BR, DG), lambda p, j: (_j0(p, j), 0)),
                     vspec, vspec,
                     pl.BlockSpec((din, H), lambda p, j: (0, 0)),
                     pl.BlockSpec((din, H), lambda p, j: (0, 0)),
                     vspec]),
        out_specs=[
            pl.BlockSpec((BR, 1), lambda p, j: (_j0(p, j), 0)),
            pl.BlockSpec((BR, H), lambda p, j: (j, 0)),
        ],
        out_shape=[jax.ShapeDtypeStruct((N, 1), _f32),
                   jax.ShapeDtypeStruct((N, H), _f32)],
        scratch_shapes=_SCRATCH,
        interpret=interpret,
    )(segA, segA, segB, segB, degp, degp, x, g0, g1, g2,
      g.reshape(1, H), be.reshape(1, H), wl, wr, b.reshape(1, H))


def _layer_call(segp, deg, h, g, be, wl, wr, b, interpret=False):
    vspec = pl.BlockSpec((1, H), lambda p, j: (0, 0))
    wspec = pl.BlockSpec((H, H), lambda p, j: (0, 0))
    return pl.pallas_call(
        _layer_body,
        grid=(2, NB),
        in_specs=(_seg_specs()
                  + [pl.BlockSpec((BR, 1), lambda p, j: (_j0(p, j), 0)),
                     pl.BlockSpec((BR, H), lambda p, j: (_j0(p, j), 0)),
                     vspec, vspec, wspec, wspec, vspec]),
        out_specs=[pl.BlockSpec((BR, H), lambda p, j: (j, 0))],
        out_shape=[jax.ShapeDtypeStruct((N, H), _f32)],
        scratch_shapes=_SCRATCH,
        interpret=interpret,
    )(segp, segp, deg, h, g.reshape(1, H), be.reshape(1, H),
      wl, wr, b.reshape(1, H))


BRH = 512
NBH = PO_LEN // BRH


def _head1_body(arr_r, bd_r, b1t_r, cd_r, b2_r, v_o):
    y = jnp.maximum(
        jnp.dot(arr_r[...].astype(jnp.bfloat16), bd_r[...],
                preferred_element_type=_f32) + b1t_r[...], 0.0)
    zv = jnp.dot(y.astype(jnp.bfloat16), cd_r[...],
                 preferred_element_type=_f32) + b2_r[...]
    zv3 = zv.reshape(BRH // 8, 8, 16)
    v_o[...] = jnp.concatenate([zv3[:, u, :] for u in range(8)], axis=1)


def _head1_call(arr, w1, b1, w2, b2, interpret=False):
    import jax.scipy.linalg as jsl
    bd = jsl.block_diag(*([w1] * 16)).astype(jnp.bfloat16)
    cd = jsl.block_diag(*([w2] * 16)).astype(jnp.bfloat16)
    b1t = jnp.tile(b1, (16,)).reshape(1, 16 * H)
    return pl.pallas_call(
        _head1_body,
        grid=(NBH,),
        in_specs=[
            pl.BlockSpec((BRH, H), lambda j: (j, 0)),
            pl.BlockSpec((H, 16 * H), lambda j: (0, 0)),
            pl.BlockSpec((1, 16 * H), lambda j: (0, 0)),
            pl.BlockSpec((16 * H, 16), lambda j: (0, 0)),
            pl.BlockSpec((1, 1), lambda j: (0, 0)),
        ],
        out_specs=[pl.BlockSpec((BRH // 8, H), lambda j: (j, 0))],
        out_shape=[jax.ShapeDtypeStruct((PO_LEN // 8, H), _f32)],
        interpret=interpret,
    )(arr, bd, b1t, cd, b2.reshape(1, 1))


def _head2_body(v_r, g_r, be_r, w1_r, b1_r, w2_r, b2_r, o_r):
    v = v_r[...]
    mu = jnp.mean(v, axis=0, keepdims=True)
    d = v - mu
    var = jnp.mean(d * d, axis=0, keepdims=True)
    f = jnp.maximum(d * lax.rsqrt(var + 1e-5) * g_r[...] + be_r[...], 0.0)
    u = jnp.maximum(jnp.dot(f, w1_r[...]) + b1_r[...], 0.0)
    o_r[...] = jnp.maximum(jnp.dot(u, w2_r[...]) + b2_r[...], 0.0)


def _head2_call(v, g, be, w1, b1, w2, b2, interpret=False):
    m = PO_LEN // 8
    return pl.pallas_call(
        _head2_body,
        out_shape=jax.ShapeDtypeStruct((m, OUT), _f32),
        interpret=interpret,
    )(v, g.reshape(1, H), be.reshape(1, H), w1, b1.reshape(1, H),
      w2, b2.reshape(1, OUT))


def kernel(x, gam0, gam1, gam2, edge_index, po, po_batch,
           conv0_Wl, conv0_Wr, conv0_b, bn0_g, bn0_b,
           conv1_Wl, conv1_Wr, conv1_b, bn1_g, bn1_b,
           conv2_Wl, conv2_Wr, conv2_b, bn2_g, bn2_b,
           conv3_Wl, conv3_Wr, conv3_b, bn3_g, bn3_b,
           mlp1_W1, mlp1_b1, mlp1_W2, mlp1_b2,
           bnf_g, bnf_b,
           mlp2_W1, mlp2_b1, mlp2_W2, mlp2_b2):
    pad = jnp.zeros((ECAP - E,), jnp.int32)
    src2d = jnp.concatenate([edge_index[0], pad]).reshape(ECAP // CH, CH)
    dst2d = jnp.concatenate([edge_index[1], pad]).reshape(ECAP // CH, CH)
    po2d = po.reshape(PO_LEN // CH, CH)
    z128 = jnp.zeros((NP, H), _f32)
    ones_rows = jnp.ones((CH, DW), _f32)
    h0a = jnp.concatenate([x, gam0], axis=1)
    h0b = jnp.concatenate([gam1, gam2], axis=1)

    seg_call = _make_seg_call()
    deg_call = _make_deg_call()
    po_call = _make_po_gather()

    degp = deg_call(dst2d, z128, ones_rows)[0]

    segA = seg_call(h0a, src2d, dst2d, z128)[0]
    segB = seg_call(h0b, src2d, dst2d, z128)[0]
    deg, h1 = _layer0_call(segA, segB, degp, x, gam0, gam1, gam2,
                           bn0_g, bn0_b, conv0_Wl, conv0_Wr, conv0_b)
    segp = seg_call(h1, src2d, dst2d, z128)[0]
    h2 = _layer_call(segp, deg, h1, bn1_g, bn1_b,
                     conv1_Wl, conv1_Wr, conv1_b)[0]
    segp = seg_call(h2, src2d, dst2d, z128)[0]
    h3 = _layer_call(segp, deg, h2, bn2_g, bn2_b,
                     conv2_Wl, conv2_Wr, conv2_b)[0]
    segp = seg_call(h3, src2d, dst2d, z128)[0]
    h4 = _layer_call(segp, deg, h3, bn3_g, bn3_b,
                     conv3_Wl, conv3_Wr, conv3_b)[0]

    arr = po_call(h4, po2d)
    v = _head1_call(arr, mlp1_W1, mlp1_b1, mlp1_W2, mlp1_b2)[0]
    return _head2_call(v, bnf_g, bnf_b, mlp2_W1, mlp2_b1, mlp2_W2, mlp2_b2)
